# baseline (speedup 1.0000x reference)
import jax
import jax.numpy as jnp
from jax.experimental import pallas as pl
from jax.experimental.pallas import tpu as pltpu

D_MODEL = 4096
VOCAB = 65024
BATCH = 64
EPS = 1e-5

TILE_V = 1536
GRID = (VOCAB + TILE_V - 1) // TILE_V


def _lm_head_kernel(h_ref, lnw_ref, w_ref, out_ref, hn_ref, bv_ref, bi_ref):
    i = pl.program_id(0)

    @pl.when(i == 0)
    def _init():
        x = h_ref[...]
        var = jnp.mean(x * x, axis=-1, keepdims=True)
        hn_ref[...] = (x * jax.lax.rsqrt(var + EPS) * lnw_ref[...]).astype(jnp.bfloat16)
        bv_ref[...] = jnp.full((BATCH, 1), -jnp.inf, dtype=jnp.float32)
        bi_ref[...] = jnp.zeros((BATCH, 1), dtype=jnp.int32)

    h = hn_ref[...]
    w = w_ref[...].astype(jnp.bfloat16)
    logits = jax.lax.dot_general(
        h, w, (((1,), (1,)), ((), ())),
        preferred_element_type=jnp.float32,
        precision=jax.lax.Precision.DEFAULT,
    )

    col = jax.lax.broadcasted_iota(jnp.int32, (BATCH, TILE_V), 1)
    logits = jnp.where(col < VOCAB - i * TILE_V, logits, -jnp.inf)
    lmax = jnp.max(logits, axis=1, keepdims=True)
    lidx = jnp.min(
        jnp.where(logits == lmax, col, jnp.iinfo(jnp.int32).max),
        axis=1, keepdims=True,
    ) + i * TILE_V

    better = lmax > bv_ref[...]
    bv_ref[...] = jnp.where(better, lmax, bv_ref[...])
    bi_ref[...] = jnp.where(better, lidx, bi_ref[...])

    @pl.when(i == GRID - 1)
    def _fin():
        out_ref[...] = bi_ref[...]


def kernel(hidden_states, ln_weight, lm_head_weight):
    lnw2d = ln_weight.reshape(1, D_MODEL)
    return pl.pallas_call(
        _lm_head_kernel,
        grid=(GRID,),
        in_specs=[
            pl.BlockSpec((BATCH, D_MODEL), lambda i: (0, 0)),
            pl.BlockSpec((1, D_MODEL), lambda i: (0, 0)),
            pl.BlockSpec((TILE_V, D_MODEL), lambda i: (i, 0)),
        ],
        out_specs=pl.BlockSpec((BATCH, 1), lambda i: (0, 0)),
        out_shape=jax.ShapeDtypeStruct((BATCH, 1), jnp.int32),
        scratch_shapes=[
            pltpu.VMEM((BATCH, D_MODEL), jnp.bfloat16),
            pltpu.VMEM((BATCH, 1), jnp.float32),
            pltpu.VMEM((BATCH, 1), jnp.int32),
        ],
        compiler_params=pltpu.CompilerParams(
            dimension_semantics=("arbitrary",),
        ),
    )(hidden_states, lnw2d, lm_head_weight)


# speedup vs baseline: 1.0119x; 1.0119x over previous
import jax
import jax.numpy as jnp
from jax.experimental import pallas as pl
from jax.experimental.pallas import tpu as pltpu

D_MODEL = 4096
VOCAB = 65024
BATCH = 64
EPS = 1e-5

TILE_V = 1024
GRID = (VOCAB + TILE_V - 1) // TILE_V


def _lm_head_kernel(h_ref, lnw_ref, w_ref, out_ref, hn_ref, bv_ref, bi_ref):
    i = pl.program_id(0)

    @pl.when(i == 0)
    def _init():
        x = h_ref[...]
        var = jnp.mean(x * x, axis=-1, keepdims=True)
        hn_ref[...] = (x * jax.lax.rsqrt(var + EPS) * lnw_ref[...]).astype(jnp.bfloat16)
        bv_ref[...] = jnp.full((BATCH, 1), -jnp.inf, dtype=jnp.float32)
        bi_ref[...] = jnp.zeros((BATCH, 1), dtype=jnp.int32)

    h = hn_ref[...]
    w = w_ref[...].astype(jnp.bfloat16)
    logits = jax.lax.dot_general(
        h, w, (((1,), (1,)), ((), ())),
        preferred_element_type=jnp.float32,
        precision=jax.lax.Precision.DEFAULT,
    )

    col = jax.lax.broadcasted_iota(jnp.int32, (BATCH, TILE_V), 1)
    logits = jnp.where(col < VOCAB - i * TILE_V, logits, -jnp.inf)
    lmax = jnp.max(logits, axis=1, keepdims=True)
    lidx = jnp.min(
        jnp.where(logits == lmax, col, jnp.iinfo(jnp.int32).max),
        axis=1, keepdims=True,
    ) + i * TILE_V

    better = lmax > bv_ref[...]
    bv_ref[...] = jnp.where(better, lmax, bv_ref[...])
    bi_ref[...] = jnp.where(better, lidx, bi_ref[...])

    @pl.when(i == GRID - 1)
    def _fin():
        out_ref[...] = bi_ref[...]


def kernel(hidden_states, ln_weight, lm_head_weight):
    lnw2d = ln_weight.reshape(1, D_MODEL)
    return pl.pallas_call(
        _lm_head_kernel,
        grid=(GRID,),
        in_specs=[
            pl.BlockSpec((BATCH, D_MODEL), lambda i: (0, 0)),
            pl.BlockSpec((1, D_MODEL), lambda i: (0, 0)),
            pl.BlockSpec((TILE_V, D_MODEL), lambda i: (i, 0)),
        ],
        out_specs=pl.BlockSpec((BATCH, 1), lambda i: (0, 0)),
        out_shape=jax.ShapeDtypeStruct((BATCH, 1), jnp.int32),
        scratch_shapes=[
            pltpu.VMEM((BATCH, D_MODEL), jnp.bfloat16),
            pltpu.VMEM((BATCH, 1), jnp.float32),
            pltpu.VMEM((BATCH, 1), jnp.int32),
        ],
        compiler_params=pltpu.CompilerParams(
            dimension_semantics=("arbitrary",),
        ),
    )(hidden_states, lnw2d, lm_head_weight)
